# Initial kernel scaffold; baseline (speedup 1.0000x reference)
import functools

import jax
import jax.numpy as jnp
from jax import lax
from jax.experimental import pallas as pl
from jax.experimental.pallas import tpu as pltpu
from jax.experimental.pallas import tpu_sc as plsc

_NC = 2
_NS = 16
_NW = _NC * _NS

_TILE = 128
_RT = 256


def _router_body(x_ref, rw_ref, idx_ref, val_ref):
    xb = x_ref[...]
    rw = rw_ref[...]
    logits = lax.dot_general(xb, rw, (((1,), (1,)), ((), ())),
                             preferred_element_type=jnp.float32)
    z = logits - jnp.max(logits, axis=1, keepdims=True)
    ez = jnp.exp(z)
    probs = ez / jnp.sum(ez, axis=1, keepdims=True)
    n_exp = probs.shape[1]
    iota = lax.broadcasted_iota(jnp.int32, probs.shape, 1)
    m1 = jnp.max(probs, axis=1, keepdims=True)
    i1 = jnp.min(jnp.where(probs == m1, iota, n_exp), axis=1, keepdims=True)
    p2 = jnp.where(iota == i1, -jnp.inf, probs)
    m2 = jnp.max(p2, axis=1, keepdims=True)
    i2 = jnp.min(jnp.where(p2 == m2, iota, n_exp), axis=1, keepdims=True)
    idx_ref[...] = jnp.concatenate([i1, i2], axis=1)
    val_ref[...] = jnp.concatenate([m1, m2], axis=1)


def _router(xf, router_w):
    t, d = xf.shape
    e = router_w.shape[0]
    return pl.pallas_call(
        _router_body,
        grid=(t // _RT,),
        in_specs=[
            pl.BlockSpec((_RT, d), lambda i: (i, 0)),
            pl.BlockSpec((e, d), lambda i: (0, 0)),
        ],
        out_specs=[
            pl.BlockSpec((_RT, 2), lambda i: (i, 0)),
            pl.BlockSpec((_RT, 2), lambda i: (i, 0)),
        ],
        out_shape=[
            jax.ShapeDtypeStruct((t, 2), jnp.int32),
            jax.ShapeDtypeStruct((t, 2), jnp.float32),
        ],
    )(xf, router_w)


def _metadata(top_idx, top_val, n_experts, n_rows):
    tk = top_idx.shape[0] * top_idx.shape[1]
    k = top_idx.shape[1]
    ids = top_idx.reshape(-1)
    gv = top_val.reshape(-1)
    tokens = (jnp.arange(tk, dtype=jnp.int32) // k).astype(jnp.int32)
    order = jnp.argsort(ids)
    ids_s = jnp.take(ids, order)
    counts = jnp.bincount(ids, length=n_experts).astype(jnp.int32)
    padded = ((counts + _TILE - 1) // _TILE) * _TILE
    pad_end = jnp.cumsum(padded)
    pad_off = pad_end - padded
    start = jnp.cumsum(counts) - counts
    pos = jnp.arange(tk, dtype=jnp.int32)
    rows_s = (jnp.take(pad_off, ids_s) + (pos - jnp.take(start, ids_s))).astype(jnp.int32)
    inv_rows = jnp.zeros((tk,), jnp.int32).at[order].set(rows_s)
    row_tok = jnp.zeros((n_rows,), jnp.int32).at[rows_s].set(jnp.take(tokens, order))
    row_gate = jnp.zeros((n_rows,), jnp.float32).at[rows_s].set(jnp.take(gv, order))
    nb = n_rows // _TILE
    block_expert = jnp.minimum(
        jnp.searchsorted(pad_end // _TILE, jnp.arange(nb, dtype=jnp.int32),
                         side='right'),
        n_experts - 1).astype(jnp.int32)
    return inv_rows, row_tok, row_gate, block_expert


def _gather_rows(xf, row_tok, n_rows):
    t, d = xf.shape
    per_w = n_rows // _NW
    ch = per_w // 2
    mesh = plsc.VectorSubcoreMesh(core_axis_name="c", subcore_axis_name="s")

    @functools.partial(
        pl.kernel, mesh=mesh,
        out_type=jax.ShapeDtypeStruct((n_rows, d), jnp.float32),
        scratch_types=[
            pltpu.VMEM((ch,), jnp.int32),
            pltpu.VMEM((ch, d), jnp.float32),
            pltpu.SemaphoreType.DMA,
        ],
    )
    def k(x_hbm, tok_hbm, out_hbm, idx_v, rows_v, sem):
        wid = lax.axis_index("s") * _NC + lax.axis_index("c")
        for c in range(per_w // ch):
            base = wid * per_w + c * ch
            pltpu.sync_copy(tok_hbm.at[pl.ds(base, ch)], idx_v)
            pltpu.async_copy(x_hbm.at[idx_v], rows_v, sem).wait()
            pltpu.sync_copy(rows_v, out_hbm.at[pl.ds(base, ch)])

    return k(xf, row_tok)


def _mlp_body(be_ref, xs_ref, wg_ref, wi_ref, bg_ref, bi_ref, wo_ref,
              bo_ref, gate_ref, out_ref):
    del be_ref
    xb = xs_ref[...]
    hg = lax.dot_general(xb, wg_ref[0], (((1,), (1,)), ((), ())),
                         preferred_element_type=jnp.float32) + bg_ref[...]
    hi = lax.dot_general(xb, wi_ref[0], (((1,), (1,)), ((), ())),
                         preferred_element_type=jnp.float32) + bi_ref[...]
    act = hg * lax.logistic(hg) * hi
    out = lax.dot_general(act, wo_ref[0], (((1,), (1,)), ((), ())),
                          preferred_element_type=jnp.float32) + bo_ref[...]
    out_ref[...] = out * gate_ref[...]


def _grouped_mlp(xs, w_in, b_in, w_out, b_out, row_gate, block_expert):
    n_rows, d = xs.shape
    e, f2, _ = w_in.shape
    f = f2 // 2
    nb = n_rows // _TILE
    grid_spec = pltpu.PrefetchScalarGridSpec(
        num_scalar_prefetch=1,
        grid=(nb,),
        in_specs=[
            pl.BlockSpec((_TILE, d), lambda i, be: (i, 0)),
            pl.BlockSpec((1, f, d), lambda i, be: (be[i], 0, 0)),
            pl.BlockSpec((1, f, d), lambda i, be: (be[i], 1, 0)),
            pl.BlockSpec((1, f), lambda i, be: (be[i], 0)),
            pl.BlockSpec((1, f), lambda i, be: (be[i], 1)),
            pl.BlockSpec((1, d, f), lambda i, be: (be[i], 0, 0)),
            pl.BlockSpec((1, d), lambda i, be: (be[i], 0)),
            pl.BlockSpec((_TILE, 1), lambda i, be: (i, 0)),
        ],
        out_specs=pl.BlockSpec((_TILE, d), lambda i, be: (i, 0)),
    )
    return pl.pallas_call(
        _mlp_body,
        grid_spec=grid_spec,
        out_shape=jax.ShapeDtypeStruct((n_rows, d), jnp.float32),
        compiler_params=pltpu.CompilerParams(
            dimension_semantics=("arbitrary",)),
    )(block_expert, xs, w_in, w_in, b_in, b_in, w_out, b_out,
      row_gate.reshape(n_rows, 1))


def _combine(out_rows, inv_rows, t):
    n_rows, d = out_rows.shape
    tpw = t // _NW
    cht = 16
    mesh = plsc.VectorSubcoreMesh(core_axis_name="c", subcore_axis_name="s")

    @functools.partial(
        pl.kernel, mesh=mesh,
        out_type=jax.ShapeDtypeStruct((t, d), jnp.float32),
        scratch_types=[
            pltpu.VMEM((2 * cht,), jnp.int32),
            pltpu.VMEM((2 * cht, d), jnp.float32),
            pltpu.VMEM((cht, d), jnp.float32),
            pltpu.SemaphoreType.DMA,
        ],
    )
    def k(rows_hbm, inv_hbm, y_hbm, idx_v, r_v, y_v, sem):
        wid = lax.axis_index("s") * _NC + lax.axis_index("c")
        for c in range(tpw // cht):
            tbase = wid * tpw + c * cht
            pltpu.sync_copy(inv_hbm.at[pl.ds(2 * tbase, 2 * cht)], idx_v)
            pltpu.async_copy(rows_hbm.at[idx_v], r_v, sem).wait()

            def body(tt, carry):
                for dc in range(d // 16):
                    sl = pl.ds(dc * 16, 16)
                    y_v[tt, sl] = r_v[2 * tt, sl] + r_v[2 * tt + 1, sl]
                return carry

            lax.fori_loop(0, cht, body, 0)
            pltpu.sync_copy(y_v, y_hbm.at[pl.ds(tbase, cht)])

    return k(out_rows, inv_rows)


def kernel(x, router_w, w_in, b_in, w_out, b_out):
    bq, sq, d = x.shape
    t = bq * sq
    e = router_w.shape[0]
    k = 2
    xf = x.reshape(t, d)

    top_idx, top_val = _router(xf, router_w)

    n_rows = t * k + e * _TILE
    inv_rows, row_tok, row_gate, block_expert = _metadata(
        top_idx, top_val, e, n_rows)

    xs = _gather_rows(xf, row_tok, n_rows)
    out_rows = _grouped_mlp(xs, w_in, b_in, w_out, b_out, row_gate,
                            block_expert)
    y = _combine(out_rows, inv_rows, t)
    return y.reshape(bq, sq, d)


# speedup vs baseline: 1.4321x; 1.4321x over previous
import functools

import jax
import jax.numpy as jnp
from jax import lax
from jax.experimental import pallas as pl
from jax.experimental.pallas import tpu as pltpu
from jax.experimental.pallas import tpu_sc as plsc

_NC = 2
_NS = 16
_NW = _NC * _NS

_TILE = 128
_RT = 256


def _router_body(x_ref, rw_ref, idx_ref, val_ref):
    xb = x_ref[...]
    rw = rw_ref[...]
    logits = lax.dot_general(xb, rw, (((1,), (1,)), ((), ())),
                             preferred_element_type=jnp.float32)
    z = logits - jnp.max(logits, axis=1, keepdims=True)
    ez = jnp.exp(z)
    probs = ez / jnp.sum(ez, axis=1, keepdims=True)
    n_exp = probs.shape[1]
    iota = lax.broadcasted_iota(jnp.int32, probs.shape, 1)
    m1 = jnp.max(probs, axis=1, keepdims=True)
    i1 = jnp.min(jnp.where(probs == m1, iota, n_exp), axis=1, keepdims=True)
    p2 = jnp.where(iota == i1, -jnp.inf, probs)
    m2 = jnp.max(p2, axis=1, keepdims=True)
    i2 = jnp.min(jnp.where(p2 == m2, iota, n_exp), axis=1, keepdims=True)
    idx_ref[...] = jnp.concatenate([i1, i2], axis=1)
    val_ref[...] = jnp.concatenate([m1, m2], axis=1)


def _router(xf, router_w):
    t, d = xf.shape
    e = router_w.shape[0]
    return pl.pallas_call(
        _router_body,
        grid=(t // _RT,),
        in_specs=[
            pl.BlockSpec((_RT, d), lambda i: (i, 0)),
            pl.BlockSpec((e, d), lambda i: (0, 0)),
        ],
        out_specs=[
            pl.BlockSpec((_RT, 2), lambda i: (i, 0)),
            pl.BlockSpec((_RT, 2), lambda i: (i, 0)),
        ],
        out_shape=[
            jax.ShapeDtypeStruct((t, 2), jnp.int32),
            jax.ShapeDtypeStruct((t, 2), jnp.float32),
        ],
    )(xf, router_w)


def _metadata(top_idx, top_val, n_experts, n_rows):
    tk = top_idx.shape[0] * top_idx.shape[1]
    k = top_idx.shape[1]
    ids = top_idx.reshape(-1)
    gv = top_val.reshape(-1)
    tokens = (jnp.arange(tk, dtype=jnp.int32) // k).astype(jnp.int32)
    order = jnp.argsort(ids)
    ids_s = jnp.take(ids, order)
    counts = jnp.bincount(ids, length=n_experts).astype(jnp.int32)
    padded = ((counts + _TILE - 1) // _TILE) * _TILE
    pad_end = jnp.cumsum(padded)
    pad_off = pad_end - padded
    start = jnp.cumsum(counts) - counts
    pos = jnp.arange(tk, dtype=jnp.int32)
    rows_s = (jnp.take(pad_off, ids_s) + (pos - jnp.take(start, ids_s))).astype(jnp.int32)
    inv_rows = jnp.zeros((tk,), jnp.int32).at[order].set(rows_s)
    row_tok = jnp.zeros((n_rows,), jnp.int32).at[rows_s].set(jnp.take(tokens, order))
    row_gate = jnp.zeros((n_rows,), jnp.float32).at[rows_s].set(jnp.take(gv, order))
    nb = n_rows // _TILE
    block_expert = jnp.minimum(
        jnp.searchsorted(pad_end // _TILE, jnp.arange(nb, dtype=jnp.int32),
                         side='right'),
        n_experts - 1).astype(jnp.int32)
    return inv_rows, row_tok, row_gate, block_expert


def _gather_rows(xf, row_tok, n_rows):
    t, d = xf.shape
    per_w = n_rows // _NW
    ch = per_w // 2
    mesh = plsc.VectorSubcoreMesh(core_axis_name="c", subcore_axis_name="s")

    @functools.partial(
        pl.kernel, mesh=mesh,
        out_type=jax.ShapeDtypeStruct((n_rows, d), jnp.float32),
        scratch_types=[
            pltpu.VMEM((ch,), jnp.int32),
            pltpu.VMEM((ch, d), jnp.float32),
            pltpu.SemaphoreType.DMA,
        ],
    )
    def k(x_hbm, tok_hbm, out_hbm, idx_v, rows_v, sem):
        wid = lax.axis_index("s") * _NC + lax.axis_index("c")
        for c in range(per_w // ch):
            base = wid * per_w + c * ch
            pltpu.sync_copy(tok_hbm.at[pl.ds(base, ch)], idx_v)
            pltpu.async_copy(x_hbm.at[idx_v], rows_v, sem).wait()
            pltpu.sync_copy(rows_v, out_hbm.at[pl.ds(base, ch)])

    return k(xf, row_tok)


def _mlp_body(be_ref, xs_ref, wg_ref, wi_ref, bg_ref, bi_ref, wo_ref,
              bo_ref, gate_ref, out_ref):
    del be_ref
    xb = xs_ref[...]
    hg = lax.dot_general(xb, wg_ref[0], (((1,), (1,)), ((), ())),
                         preferred_element_type=jnp.float32) + bg_ref[0]
    hi = lax.dot_general(xb, wi_ref[0], (((1,), (1,)), ((), ())),
                         preferred_element_type=jnp.float32) + bi_ref[0]
    act = hg * lax.logistic(hg) * hi
    out = lax.dot_general(act, wo_ref[0], (((1,), (1,)), ((), ())),
                          preferred_element_type=jnp.float32) + bo_ref[0]
    out_ref[...] = out * gate_ref[...]


def _grouped_mlp(xs, w_in, b_in, w_out, b_out, row_gate, block_expert):
    n_rows, d = xs.shape
    e, f2, _ = w_in.shape
    f = f2 // 2
    nb = n_rows // _TILE
    grid_spec = pltpu.PrefetchScalarGridSpec(
        num_scalar_prefetch=1,
        grid=(nb,),
        in_specs=[
            pl.BlockSpec((_TILE, d), lambda i, be: (i, 0)),
            pl.BlockSpec((1, f, d), lambda i, be: (be[i], 0, 0)),
            pl.BlockSpec((1, f, d), lambda i, be: (be[i], 1, 0)),
            pl.BlockSpec((1, 1, f), lambda i, be: (2 * be[i], 0, 0)),
            pl.BlockSpec((1, 1, f), lambda i, be: (2 * be[i] + 1, 0, 0)),
            pl.BlockSpec((1, d, f), lambda i, be: (be[i], 0, 0)),
            pl.BlockSpec((1, 1, d), lambda i, be: (be[i], 0, 0)),
            pl.BlockSpec((_TILE, 1), lambda i, be: (i, 0)),
        ],
        out_specs=pl.BlockSpec((_TILE, d), lambda i, be: (i, 0)),
    )
    return pl.pallas_call(
        _mlp_body,
        grid_spec=grid_spec,
        out_shape=jax.ShapeDtypeStruct((n_rows, d), jnp.float32),
        compiler_params=pltpu.CompilerParams(
            dimension_semantics=("arbitrary",)),
    )(block_expert, xs, w_in, w_in, b_in.reshape(2 * e, 1, f),
      b_in.reshape(2 * e, 1, f), w_out, b_out.reshape(e, 1, d),
      row_gate.reshape(n_rows, 1))


def _combine(out_rows, inv_rows, t):
    n_rows, d = out_rows.shape
    tpw = t // _NW
    cht = 16
    mesh = plsc.VectorSubcoreMesh(core_axis_name="c", subcore_axis_name="s")

    @functools.partial(
        pl.kernel, mesh=mesh,
        out_type=jax.ShapeDtypeStruct((t, d), jnp.float32),
        scratch_types=[
            pltpu.VMEM((2 * cht,), jnp.int32),
            pltpu.VMEM((2 * cht, d), jnp.float32),
            pltpu.VMEM((cht, d), jnp.float32),
            pltpu.SemaphoreType.DMA,
        ],
    )
    def k(rows_hbm, inv_hbm, y_hbm, idx_v, r_v, y_v, sem):
        wid = lax.axis_index("s") * _NC + lax.axis_index("c")
        for c in range(tpw // cht):
            tbase = wid * tpw + c * cht
            pltpu.sync_copy(inv_hbm.at[pl.ds(2 * tbase, 2 * cht)], idx_v)
            pltpu.async_copy(rows_hbm.at[idx_v], r_v, sem).wait()

            def body(tt, carry):
                for dc in range(d // 16):
                    sl = pl.ds(dc * 16, 16)
                    y_v[tt, sl] = r_v[2 * tt, sl] + r_v[2 * tt + 1, sl]
                return carry

            lax.fori_loop(0, cht, body, 0)
            pltpu.sync_copy(y_v, y_hbm.at[pl.ds(tbase, cht)])

    return k(out_rows, inv_rows)


def kernel(x, router_w, w_in, b_in, w_out, b_out):
    bq, sq, d = x.shape
    t = bq * sq
    e = router_w.shape[0]
    k = 2
    xf = x.reshape(t, d)

    top_idx, top_val = _router(xf, router_w)

    n_rows = t * k + e * _TILE
    inv_rows, row_tok, row_gate, block_expert = _metadata(
        top_idx, top_val, e, n_rows)

    xs = _gather_rows(xf, row_tok, n_rows)
    out_rows = _grouped_mlp(xs, w_in, b_in, w_out, b_out, row_gate,
                            block_expert)
    y = _combine(out_rows, inv_rows, t)
    return y.reshape(bq, sq, d)
